# baseline (speedup 1.0000x reference)
import jax
import jax.numpy as jnp
from jax import lax
from jax.experimental import pallas as pl
from jax.experimental.pallas import tpu as pltpu
from jax.experimental.pallas import tpu_sc as plsc

_NUM_STEPS = 1000
_DIM = 128
_BATCH = 16384

_info = plsc.get_sparse_core_info()
_NC, _NS = _info.num_cores, _info.num_subcores
_NW = _NC * _NS
_BPW = _BATCH // _NW
_CHUNK = 128
_NCHUNK = _BPW // _CHUNK

_TROWS = 64


def _gather_kernel(table_hbm, idx_hbm, out_hbm, idx_v, rows_v, tab_s, sem, isem):
    cid = lax.axis_index("c")
    sid = lax.axis_index("s")
    wid = sid * _NC + cid
    base = wid * _BPW
    idx_copy = pltpu.async_copy(idx_hbm.at[wid], idx_v, isem)
    @pl.when(sid < 15)
    def _stage():
        pltpu.sync_copy(
            table_hbm.at[pl.ds(sid * _TROWS, _TROWS)],
            tab_s.at[pl.ds(sid * _TROWS, _TROWS)],
        )

    @pl.when(sid == 15)
    def _stage_tail():
        pltpu.sync_copy(
            table_hbm.at[pl.ds(15 * _TROWS, _NUM_STEPS - 15 * _TROWS)],
            tab_s.at[pl.ds(15 * _TROWS, _NUM_STEPS - 15 * _TROWS)],
        )

    idx_copy.wait()
    plsc.subcore_barrier()
    gathers = []
    for j in range(_NCHUNK):
        gathers.append(
            pltpu.async_copy(
                tab_s.at[idx_v.at[j]],
                rows_v.at[pl.ds(j * _CHUNK, _CHUNK)],
                sem,
            )
        )
    for g in gathers:
        g.wait()
    pltpu.sync_copy(rows_v, out_hbm.at[pl.ds(base, _BPW)])


@jax.jit
def _lookup(input, table):
    idx3 = input.reshape(_NW, _NCHUNK, _CHUNK)
    mesh = plsc.VectorSubcoreMesh(core_axis_name="c", subcore_axis_name="s")
    return pl.kernel(
        _gather_kernel,
        mesh=mesh,
        out_type=jax.ShapeDtypeStruct((_BATCH, _DIM), jnp.float32),
        scratch_types=[
            pltpu.VMEM((_NCHUNK, _CHUNK), jnp.int32),
            pltpu.VMEM((_BPW, _DIM), jnp.float32),
            pltpu.VMEM_SHARED((_NUM_STEPS, _DIM), jnp.float32),
            pltpu.SemaphoreType.DMA,
            pltpu.SemaphoreType.DMA,
        ],
    )(table, idx3)


def kernel(input, table):
    return _lookup(input, table)


# speedup vs baseline: 1.0026x; 1.0026x over previous
import jax
import jax.numpy as jnp
from jax import lax
from jax.experimental import pallas as pl
from jax.experimental.pallas import tpu as pltpu
from jax.experimental.pallas import tpu_sc as plsc

_NUM_STEPS = 1000
_DIM = 128
_BATCH = 16384

_info = plsc.get_sparse_core_info()
_NC, _NS = _info.num_cores, _info.num_subcores
_NW = _NC * _NS
_BPW = _BATCH // _NW
_CHUNK = 128
_NCHUNK = _BPW // _CHUNK

_TROWS = 64


def _gather_kernel(table_hbm, idx_hbm, out_hbm, idx_v, rows_v, tab_s, sem, isem):
    cid = lax.axis_index("c")
    sid = lax.axis_index("s")
    wid = sid * _NC + cid
    base = wid * _BPW
    idx_copy = pltpu.async_copy(idx_hbm.at[wid], idx_v, isem)
    @pl.when(sid < 15)
    def _stage():
        pltpu.sync_copy(
            table_hbm.at[pl.ds(sid * _TROWS, _TROWS)],
            tab_s.at[pl.ds(sid * _TROWS, _TROWS)],
        )

    @pl.when(sid == 15)
    def _stage_tail():
        pltpu.sync_copy(
            table_hbm.at[pl.ds(15 * _TROWS, _NUM_STEPS - 15 * _TROWS)],
            tab_s.at[pl.ds(15 * _TROWS, _NUM_STEPS - 15 * _TROWS)],
        )

    idx_copy.wait()
    plsc.subcore_barrier()
    gathers = []
    for j in range(_NCHUNK):
        gathers.append(
            pltpu.async_copy(
                tab_s.at[idx_v.at[j]],
                rows_v.at[pl.ds(j * _CHUNK, _CHUNK)],
                sem,
            )
        )
    for g in gathers:
        g.wait()
    pltpu.sync_copy(rows_v, out_hbm.at[pl.ds(base, _BPW)])


@jax.jit
def _lookup(input, table):
    idx3 = input.astype(jnp.int32).reshape(_NW, _NCHUNK, _CHUNK)
    mesh = plsc.VectorSubcoreMesh(core_axis_name="c", subcore_axis_name="s")
    return pl.kernel(
        _gather_kernel,
        mesh=mesh,
        out_type=jax.ShapeDtypeStruct((_BATCH, _DIM), jnp.float32),
        scratch_types=[
            pltpu.VMEM((_NCHUNK, _CHUNK), jnp.int32),
            pltpu.VMEM((_BPW, _DIM), jnp.float32),
            pltpu.VMEM_SHARED((_NUM_STEPS, _DIM), jnp.float32),
            pltpu.SemaphoreType.DMA,
            pltpu.SemaphoreType.DMA,
        ],
    )(table, idx3)


def kernel(input, table):
    return _lookup(input, table)
